# Initial kernel scaffold; baseline (speedup 1.0000x reference)
import jax
import jax.numpy as jnp
from jax.experimental import pallas as pl


def kernel(points, hash_table):
    raise NotImplementedError("write your pallas kernel here")


# speedup vs baseline: 3.4978x; 3.4978x over previous
import functools
import math

import jax
import jax.numpy as jnp
from jax import lax
from jax.experimental import pallas as pl
from jax.experimental.pallas import tpu as pltpu
from jax.experimental.pallas import tpu_sc as plsc

NUM_LEVELS = 16
TABLE_SIZE = 2 ** 19
FEATURE_DIMS = 2
MIN_RES = 16
N_POINTS = 524288

NC = 2
NS = 16
NW = NC * NS
PTS_PER_W = N_POINTS // NW
C = 1024
NCHUNK = PTS_PER_W // C
G = C // 16
ROWS = 8 * C
DMA_B = 128
NDMA = ROWS // DMA_B

P1 = 2654435761
P2 = 805459861
MASK = TABLE_SIZE - 1


def _body(xs_hbm, ys_hbm, zs_hbm, table_hbm, out_hbm, x_v, y_v, z_v, w_v,
          idx_v, rows_v, out_v, sem):
    wid = lax.axis_index("s") * NC + lax.axis_index("c")
    lanes = lax.iota(jnp.int32, 16)
    lanes8 = lanes * 8

    def chunk_body(ci, _):
        base = wid * PTS_PER_W + ci * C
        pltpu.sync_copy(xs_hbm.at[pl.ds(base, C)], x_v)
        pltpu.sync_copy(ys_hbm.at[pl.ds(base, C)], y_v)
        pltpu.sync_copy(zs_hbm.at[pl.ds(base, C)], z_v)

        def level_body(l, res_f):
            loff = l * TABLE_SIZE

            def hash_body(g, _):
                p0 = g * 16
                xv = x_v[pl.ds(p0, 16)]
                yv = y_v[pl.ds(p0, 16)]
                zv = z_v[pl.ds(p0, 16)]
                sx = xv * res_f + 0.5
                sy = yv * res_f + 0.5
                sz = zv * res_f + 0.5
                fx = sx.astype(jnp.int32)
                fy = sy.astype(jnp.int32)
                fz = sz.astype(jnp.int32)
                w_v[0, pl.ds(p0, 16)] = sx - fx.astype(jnp.float32)
                w_v[1, pl.ds(p0, 16)] = sy - fy.astype(jnp.float32)
                w_v[2, pl.ds(p0, 16)] = sz - fz.astype(jnp.float32)
                p1 = jnp.uint32(P1)
                p2 = jnp.uint32(P2)
                x0 = fx.astype(jnp.uint32)
                x1 = x0 + 1
                ya = fy.astype(jnp.uint32) * p1
                yb = ya + p1
                za = fz.astype(jnp.uint32) * p2
                zb = za + p2
                t00 = x0 ^ ya
                t10 = x1 ^ ya
                t01 = x0 ^ yb
                t11 = x1 ^ yb
                pos = g * 128 + lanes8
                for c in range(8):
                    t = (t00, t10, t01, t11)[c & 3]
                    zc = za if c < 4 else zb
                    h = ((t ^ zc) & MASK).astype(jnp.int32) + loff
                    plsc.store_scatter(idx_v, [pos + c], h)
                cpy = pltpu.make_async_copy(
                    table_hbm.at[idx_v.at[pl.ds(g * 128, 128)]],
                    rows_v.at[pl.ds(g * 128, 128)], sem)
                cpy.start()
                return 0

            lax.fori_loop(0, G, hash_body, 0)

            def drain_body(j, _):
                pltpu.make_async_copy(
                    table_hbm.at[idx_v.at[pl.ds(j * 128, 128)]],
                    rows_v.at[pl.ds(j * 128, 128)], sem).wait()
                return 0

            lax.fori_loop(0, G, drain_body, 0)

            def interp_body(g, _):
                p0 = g * 16
                wx = w_v[0, pl.ds(p0, 16)]
                wy = w_v[1, pl.ds(p0, 16)]
                wz = w_v[2, pl.ds(p0, 16)]
                omx = 1.0 - wx
                omy = 1.0 - wy
                omz = 1.0 - wz
                rbase = g * 128 + lanes8
                zeros = jnp.zeros((16,), jnp.int32)
                ones = zeros + 1
                f0 = []
                f1 = []
                for c in range(8):
                    ridx = rbase + c
                    f0.append(plsc.load_gather(rows_v, [ridx, zeros]))
                    f1.append(plsc.load_gather(rows_v, [ridx, ones]))
                pvec = p0 + lanes
                for f, ff in ((0, f0), (1, f1)):
                    a = ff[0] * omx + ff[1] * wx
                    b = ff[2] * omx + ff[3] * wx
                    cc = ff[4] * omx + ff[5] * wx
                    d = ff[6] * omx + ff[7] * wx
                    ab = a * omy + b * wy
                    cd = cc * omy + d * wy
                    e = ab * omz + cd * wz
                    plsc.store_scatter(out_v, [pvec * 32 + 2 * l + f], e)
                return 0

            lax.fori_loop(0, G, interp_body, 0)
            return res_f * 2.0

        lax.fori_loop(0, NUM_LEVELS, level_body, jnp.float32(MIN_RES))
        pltpu.sync_copy(out_v, out_hbm.at[pl.ds(base * 32, C * 32)])
        return 0

    lax.fori_loop(0, NCHUNK, chunk_body, 0)


@jax.jit
def _encode(xs, ys, zs, table):
    mesh = plsc.VectorSubcoreMesh(core_axis_name="c", subcore_axis_name="s")
    kern = functools.partial(
        pl.kernel,
        out_type=jax.ShapeDtypeStruct((N_POINTS * 32,), jnp.float32),
        mesh=mesh,
        compiler_params=pltpu.CompilerParams(needs_layout_passes=False,
                                             use_tc_tiling_on_sc=False),
        scratch_types=[
            pltpu.VMEM((C,), jnp.float32),
            pltpu.VMEM((C,), jnp.float32),
            pltpu.VMEM((C,), jnp.float32),
            pltpu.VMEM((3, C), jnp.float32),
            pltpu.VMEM((ROWS,), jnp.int32),
            pltpu.VMEM((ROWS, 2), jnp.float32),
            pltpu.VMEM((C * 32,), jnp.float32),
            pltpu.SemaphoreType.DMA,
        ],
    )(_body)
    return kern(xs, ys, zs, table)


def kernel(points, hash_table):
    xs = points[:, 0]
    ys = points[:, 1]
    zs = points[:, 2]
    out = _encode(xs, ys, zs, hash_table)
    return out.reshape(N_POINTS, 32)
